# baseline (device time: 75606 ns/iter reference)
import jax
import jax.numpy as jnp
from jax import lax
from jax.experimental import pallas as pl
from jax.experimental.pallas import tpu as pltpu

N_DEV = 8
M_PER = 512
N_OUT = 2048

GROUPS = (
    ((1, 3, 4), 0, 768),
    ((3, 4, 1), 768, 1408),
    ((4, 1, 3), 1408, 2048),
)

COMPUTE_ORDER = (2, 7, 5, 6, 1, 3, 4, 0)


def _gelu(y):
    c = 0.7978845608028654
    return 0.5 * y * (1.0 + jnp.tanh(c * (y + 0.044715 * y * y * y)))


def kernel(x, w_mat):
    def body(x_ref, w_ref, out_ref, acc0, acc1, acc2, rcv0, rcv1, rcv2,
             xbf_ref, wbf_ref, ssems, rsems):
        accs = (acc0, acc1, acc2)
        rcvs = (rcv0, rcv1, rcv2)
        my = lax.axis_index("i")

        xbf_ref[:, :] = x_ref[:, :].astype(jnp.bfloat16)
        wbf_ref[:, :] = w_ref[:, :].astype(jnp.bfloat16)

        barrier_sem = pltpu.get_barrier_semaphore()
        for mask in (1, 3, 4):
            pl.semaphore_signal(
                barrier_sem, inc=1,
                device_id=(my ^ mask,), device_id_type=pl.DeviceIdType.MESH,
            )
        pl.semaphore_wait(barrier_sem, 3)

        def rows(c):
            return pl.ds(c * M_PER, M_PER)

        def mk(g, k, src_chunk, partner_mask, cs, ce):
            return pltpu.make_async_remote_copy(
                src_ref=accs[g].at[rows(src_chunk), :],
                dst_ref=rcvs[g].at[k],
                send_sem=ssems.at[g, k],
                recv_sem=rsems.at[g, k],
                device_id=(my ^ partner_mask,),
                device_id_type=pl.DeviceIdType.MESH,
            )

        def send_lists(m1, m2, m3):
            s1 = (m1 ^ m2, m1 ^ m2 ^ m3, m1 ^ m3, m1)
            s2 = (m2 ^ m3, m2)
            s3 = (m3,)
            return s1, s2, s3

        issued = [0, 0, 0]
        descs = [[None] * 7, [None] * 7, [None] * 7]
        done = set()
        for cm in COMPUTE_ORDER:
            c = my ^ cm
            p = jnp.dot(
                xbf_ref[rows(c), :], wbf_ref[:, :],
                preferred_element_type=jnp.float32,
            ).astype(jnp.bfloat16)
            for g, (_, cs, ce) in enumerate(GROUPS):
                accs[g][rows(c), :] = p[:, cs:ce]
            done.add(cm)
            for g, ((m1, m2, m3), cs, ce) in enumerate(GROUPS):
                s1, _, _ = send_lists(m1, m2, m3)
                while issued[g] < 4 and s1[issued[g]] in done:
                    k = issued[g]
                    d = mk(g, k, my ^ s1[k], m1, cs, ce)
                    d.start()
                    descs[g][k] = d
                    issued[g] += 1

        def add_slot(g, k, recv_mask):
            r = rows(my ^ recv_mask)
            val = (accs[g][r, :].astype(jnp.float32)
                   + rcvs[g][k, :, :].astype(jnp.float32))
            accs[g][r, :] = val.astype(jnp.bfloat16)

        for k in range(4):
            for g, ((m1, m2, m3), cs, ce) in enumerate(GROUPS):
                s1, s2, s3 = send_lists(m1, m2, m3)
                descs[g][k].wait_recv()
                add_slot(g, k, s1[k] ^ m1)
                if k == 1:
                    descs[(g + 1) % 3][3].wait_send()
                    for j, sm in enumerate(s2):
                        d = mk(g, 4 + j, my ^ sm, m2, cs, ce)
                        d.start()
                        descs[g][4 + j] = d
        for j in range(2):
            for g, ((m1, m2, m3), cs, ce) in enumerate(GROUPS):
                s1, s2, s3 = send_lists(m1, m2, m3)
                descs[g][4 + j].wait_recv()
                add_slot(g, 4 + j, s2[j] ^ m2)
                if j == 0:
                    descs[(g + 1) % 3][5].wait_send()
                    d = mk(g, 6, my ^ s3[0], m3, cs, ce)
                    d.start()
                    descs[g][6] = d
        for g, ((m1, m2, m3), cs, ce) in enumerate(GROUPS):
            descs[g][6].wait_recv()
            final = (accs[g][rows(my), :].astype(jnp.float32)
                     + rcvs[g][6, :, :].astype(jnp.float32))
            out_ref[:, cs:ce] = _gelu(final)

        for g in range(3):
            for k in range(7):
                if k not in (3, 5):
                    descs[g][k].wait_send()

    return pl.pallas_call(
        body,
        out_shape=jax.ShapeDtypeStruct((M_PER, N_OUT), jnp.float32),
        in_specs=[
            pl.BlockSpec(memory_space=pltpu.VMEM),
            pl.BlockSpec(memory_space=pltpu.VMEM),
        ],
        out_specs=pl.BlockSpec(memory_space=pltpu.VMEM),
        scratch_shapes=[
            pltpu.VMEM((N_DEV * M_PER, GROUPS[0][2] - GROUPS[0][1]),
                       jnp.bfloat16),
            pltpu.VMEM((N_DEV * M_PER, GROUPS[1][2] - GROUPS[1][1]),
                       jnp.bfloat16),
            pltpu.VMEM((N_DEV * M_PER, GROUPS[2][2] - GROUPS[2][1]),
                       jnp.bfloat16),
            pltpu.VMEM((7, M_PER, GROUPS[0][2] - GROUPS[0][1]),
                       jnp.bfloat16),
            pltpu.VMEM((7, M_PER, GROUPS[1][2] - GROUPS[1][1]),
                       jnp.bfloat16),
            pltpu.VMEM((7, M_PER, GROUPS[2][2] - GROUPS[2][1]),
                       jnp.bfloat16),
            pltpu.VMEM((N_DEV * M_PER, M_PER), jnp.bfloat16),
            pltpu.VMEM((M_PER, N_OUT), jnp.bfloat16),
            pltpu.SemaphoreType.DMA((3, 7)),
            pltpu.SemaphoreType.DMA((3, 7)),
        ],
        compiler_params=pltpu.CompilerParams(
            collective_id=0, vmem_limit_bytes=100 * 1024 * 1024
        ),
    )(x, w_mat)


# device time: 74576 ns/iter; 1.0138x vs baseline; 1.0138x over previous
import jax
import jax.numpy as jnp
from jax import lax
from jax.experimental import pallas as pl
from jax.experimental.pallas import tpu as pltpu

N_DEV = 8
M_PER = 512
N_OUT = 2048

GROUPS = (
    ((1, 3, 4), 0, 768),
    ((3, 4, 1), 768, 1408),
    ((4, 1, 3), 1408, 2048),
)

COMPUTE_ORDER = (2, 7, 5, 6, 1, 3, 4, 0)


def _gelu(y):
    c = 0.7978845608028654
    return 0.5 * y * (1.0 + jnp.tanh(c * (y + 0.044715 * y * y * y)))


def kernel(x, w_mat):
    def body(x_ref, w_ref, out_ref, acc0, acc1, acc2, rcv0, rcv1, rcv2,
             xbf_ref, wbf_ref, ssems, rsems):
        accs = (acc0, acc1, acc2)
        rcvs = (rcv0, rcv1, rcv2)
        my = lax.axis_index("i")

        xbf_ref[:, :] = x_ref[:, :].astype(jnp.bfloat16)
        wbf_ref[:, :] = w_ref[:, :].astype(jnp.bfloat16)

        barrier_sem = pltpu.get_barrier_semaphore()
        for mask in (1, 3, 4):
            pl.semaphore_signal(
                barrier_sem, inc=1,
                device_id=(my ^ mask,), device_id_type=pl.DeviceIdType.MESH,
            )
        pl.semaphore_wait(barrier_sem, 3)

        def rows(c):
            return pl.ds(c * M_PER, M_PER)

        def mk(g, k, src_chunk, partner_mask, cs, ce):
            return pltpu.make_async_remote_copy(
                src_ref=accs[g].at[rows(src_chunk), :],
                dst_ref=rcvs[g].at[k],
                send_sem=ssems.at[g, k],
                recv_sem=rsems.at[g, k],
                device_id=(my ^ partner_mask,),
                device_id_type=pl.DeviceIdType.MESH,
            )

        def send_lists(m1, m2, m3):
            s1 = (m1 ^ m2, m1 ^ m2 ^ m3, m1 ^ m3, m1)
            s2 = (m2 ^ m3, m2)
            s3 = (m3,)
            return s1, s2, s3

        issued = [0, 0, 0]
        descs = [[None] * 7, [None] * 7, [None] * 7]
        done = set()
        for cm in COMPUTE_ORDER:
            c = my ^ cm
            p = jnp.dot(
                xbf_ref[rows(c), :], wbf_ref[:, :],
                preferred_element_type=jnp.float32,
            ).astype(jnp.bfloat16)
            for g, (_, cs, ce) in enumerate(GROUPS):
                accs[g][rows(c), :] = p[:, cs:ce]
            done.add(cm)
            for g, ((m1, m2, m3), cs, ce) in enumerate(GROUPS):
                s1, _, _ = send_lists(m1, m2, m3)
                while issued[g] < 4 and s1[issued[g]] in done:
                    k = issued[g]
                    d = mk(g, k, my ^ s1[k], m1, cs, ce)
                    d.start()
                    descs[g][k] = d
                    issued[g] += 1

        def add_slot(g, k, recv_mask):
            r = rows(my ^ recv_mask)
            val = (accs[g][r, :].astype(jnp.float32)
                   + rcvs[g][k, :, :].astype(jnp.float32))
            accs[g][r, :] = val.astype(jnp.bfloat16)

        for k in range(4):
            for g, ((m1, m2, m3), cs, ce) in enumerate(GROUPS):
                s1, s2, s3 = send_lists(m1, m2, m3)
                descs[g][k].wait_recv()
                add_slot(g, k, s1[k] ^ m1)
                if k == 1:
                    for j, sm in enumerate(s2):
                        d = mk(g, 4 + j, my ^ sm, m2, cs, ce)
                        d.start()
                        descs[g][4 + j] = d
        for j in range(2):
            for g, ((m1, m2, m3), cs, ce) in enumerate(GROUPS):
                s1, s2, s3 = send_lists(m1, m2, m3)
                descs[g][4 + j].wait_recv()
                add_slot(g, 4 + j, s2[j] ^ m2)
                if j == 0:
                    d = mk(g, 6, my ^ s3[0], m3, cs, ce)
                    d.start()
                    descs[g][6] = d
        for g, ((m1, m2, m3), cs, ce) in enumerate(GROUPS):
            descs[g][6].wait_recv()
            final = (accs[g][rows(my), :].astype(jnp.float32)
                     + rcvs[g][6, :, :].astype(jnp.float32))
            out_ref[:, cs:ce] = _gelu(final)

        for g in range(3):
            for k in range(7):
                descs[g][k].wait_send()

    return pl.pallas_call(
        body,
        out_shape=jax.ShapeDtypeStruct((M_PER, N_OUT), jnp.float32),
        in_specs=[
            pl.BlockSpec(memory_space=pltpu.VMEM),
            pl.BlockSpec(memory_space=pltpu.VMEM),
        ],
        out_specs=pl.BlockSpec(memory_space=pltpu.VMEM),
        scratch_shapes=[
            pltpu.VMEM((N_DEV * M_PER, GROUPS[0][2] - GROUPS[0][1]),
                       jnp.bfloat16),
            pltpu.VMEM((N_DEV * M_PER, GROUPS[1][2] - GROUPS[1][1]),
                       jnp.bfloat16),
            pltpu.VMEM((N_DEV * M_PER, GROUPS[2][2] - GROUPS[2][1]),
                       jnp.bfloat16),
            pltpu.VMEM((7, M_PER, GROUPS[0][2] - GROUPS[0][1]),
                       jnp.bfloat16),
            pltpu.VMEM((7, M_PER, GROUPS[1][2] - GROUPS[1][1]),
                       jnp.bfloat16),
            pltpu.VMEM((7, M_PER, GROUPS[2][2] - GROUPS[2][1]),
                       jnp.bfloat16),
            pltpu.VMEM((N_DEV * M_PER, M_PER), jnp.bfloat16),
            pltpu.VMEM((M_PER, N_OUT), jnp.bfloat16),
            pltpu.SemaphoreType.DMA((3, 7)),
            pltpu.SemaphoreType.DMA((3, 7)),
        ],
        compiler_params=pltpu.CompilerParams(
            collective_id=0, vmem_limit_bytes=100 * 1024 * 1024
        ),
    )(x, w_mat)


# device time: 74165 ns/iter; 1.0194x vs baseline; 1.0055x over previous
import jax
import jax.numpy as jnp
from jax import lax
from jax.experimental import pallas as pl
from jax.experimental.pallas import tpu as pltpu

N_DEV = 8
M_PER = 512
N_OUT = 2048

GROUPS = (
    ((1, 3, 4), 0, 768),
    ((3, 4, 1), 768, 1408),
    ((4, 1, 3), 1408, 2048),
)

COMPUTE_ORDER = (2, 7, 5, 6, 1, 3, 4, 0)


def _gelu(y):
    c = 0.7978845608028654
    return 0.5 * y * (1.0 + jnp.tanh(c * (y + 0.044715 * y * y * y)))


def kernel(x, w_mat):
    def body(x_ref, w_ref, out_ref, acc0, acc1, acc2, rcv0, rcv1, rcv2,
             wbf_ref, ssems, rsems):
        accs = (acc0, acc1, acc2)
        rcvs = (rcv0, rcv1, rcv2)
        my = lax.axis_index("i")

        wbf_ref[:, :] = w_ref[:, :].astype(jnp.bfloat16)

        barrier_sem = pltpu.get_barrier_semaphore()
        for mask in (1, 3, 4):
            pl.semaphore_signal(
                barrier_sem, inc=1,
                device_id=(my ^ mask,), device_id_type=pl.DeviceIdType.MESH,
            )
        pl.semaphore_wait(barrier_sem, 3)

        def rows(c):
            return pl.ds(c * M_PER, M_PER)

        def mk(g, k, src_chunk, partner_mask, cs, ce):
            return pltpu.make_async_remote_copy(
                src_ref=accs[g].at[rows(src_chunk), :],
                dst_ref=rcvs[g].at[k],
                send_sem=ssems.at[g, k],
                recv_sem=rsems.at[g, k],
                device_id=(my ^ partner_mask,),
                device_id_type=pl.DeviceIdType.MESH,
            )

        def send_lists(m1, m2, m3):
            s1 = (m1 ^ m2, m1 ^ m2 ^ m3, m1 ^ m3, m1)
            s2 = (m2 ^ m3, m2)
            s3 = (m3,)
            return s1, s2, s3

        issued = [0, 0, 0]
        descs = [[None] * 7, [None] * 7, [None] * 7]
        done = set()
        for cm in COMPUTE_ORDER:
            c = my ^ cm
            p = jnp.dot(
                x_ref[rows(c), :].astype(jnp.bfloat16), wbf_ref[:, :],
                preferred_element_type=jnp.float32,
            ).astype(jnp.bfloat16)
            for g, (_, cs, ce) in enumerate(GROUPS):
                accs[g][rows(c), :] = p[:, cs:ce]
            done.add(cm)
            for g, ((m1, m2, m3), cs, ce) in enumerate(GROUPS):
                s1, _, _ = send_lists(m1, m2, m3)
                while issued[g] < 4 and s1[issued[g]] in done:
                    k = issued[g]
                    d = mk(g, k, my ^ s1[k], m1, cs, ce)
                    d.start()
                    descs[g][k] = d
                    issued[g] += 1

        def add_slot(g, k, recv_mask):
            r = rows(my ^ recv_mask)
            val = (accs[g][r, :].astype(jnp.float32)
                   + rcvs[g][k, :, :].astype(jnp.float32))
            accs[g][r, :] = val.astype(jnp.bfloat16)

        for k in range(4):
            for g, ((m1, m2, m3), cs, ce) in enumerate(GROUPS):
                s1, s2, s3 = send_lists(m1, m2, m3)
                descs[g][k].wait_recv()
                add_slot(g, k, s1[k] ^ m1)
                if k == 1:
                    for j, sm in enumerate(s2):
                        d = mk(g, 4 + j, my ^ sm, m2, cs, ce)
                        d.start()
                        descs[g][4 + j] = d
        for j in range(2):
            for g, ((m1, m2, m3), cs, ce) in enumerate(GROUPS):
                s1, s2, s3 = send_lists(m1, m2, m3)
                descs[g][4 + j].wait_recv()
                add_slot(g, 4 + j, s2[j] ^ m2)
                if j == 0:
                    d = mk(g, 6, my ^ s3[0], m3, cs, ce)
                    d.start()
                    descs[g][6] = d
        for g, ((m1, m2, m3), cs, ce) in enumerate(GROUPS):
            descs[g][6].wait_recv()
            final = (accs[g][rows(my), :].astype(jnp.float32)
                     + rcvs[g][6, :, :].astype(jnp.float32))
            out_ref[:, cs:ce] = _gelu(final)

        for g in range(3):
            for k in range(7):
                descs[g][k].wait_send()

    return pl.pallas_call(
        body,
        out_shape=jax.ShapeDtypeStruct((M_PER, N_OUT), jnp.float32),
        in_specs=[
            pl.BlockSpec(memory_space=pltpu.VMEM),
            pl.BlockSpec(memory_space=pltpu.VMEM),
        ],
        out_specs=pl.BlockSpec(memory_space=pltpu.VMEM),
        scratch_shapes=[
            pltpu.VMEM((N_DEV * M_PER, GROUPS[0][2] - GROUPS[0][1]),
                       jnp.bfloat16),
            pltpu.VMEM((N_DEV * M_PER, GROUPS[1][2] - GROUPS[1][1]),
                       jnp.bfloat16),
            pltpu.VMEM((N_DEV * M_PER, GROUPS[2][2] - GROUPS[2][1]),
                       jnp.bfloat16),
            pltpu.VMEM((7, M_PER, GROUPS[0][2] - GROUPS[0][1]),
                       jnp.bfloat16),
            pltpu.VMEM((7, M_PER, GROUPS[1][2] - GROUPS[1][1]),
                       jnp.bfloat16),
            pltpu.VMEM((7, M_PER, GROUPS[2][2] - GROUPS[2][1]),
                       jnp.bfloat16),
            pltpu.VMEM((M_PER, N_OUT), jnp.bfloat16),
            pltpu.SemaphoreType.DMA((3, 7)),
            pltpu.SemaphoreType.DMA((3, 7)),
        ],
        compiler_params=pltpu.CompilerParams(
            collective_id=0, vmem_limit_bytes=100 * 1024 * 1024
        ),
    )(x, w_mat)
